# initial kernel scaffold (unmeasured)
import jax
import jax.numpy as jnp
from jax import lax
from jax.experimental import pallas as pl
from jax.experimental.pallas import tpu as pltpu

N_DEV = 4
B = 2
SQ = 512
DM = 768
NH = 8
DH = 64
HD = NH * DH
WIN = 128
KVU = SQ + WIN


def kernel(x, Wq, K_ext, V_ext, Wo):
    k2 = K_ext.reshape(B, SQ, N_DEV * HD)
    v2 = V_ext.reshape(B, SQ, N_DEV * HD)

    def body(x_ref, wq_ref, k_ref, v_ref, wo_ref, out_ref,
             kg, vg, acc, ksend, vsend, csend, kvrecv, accrecv, locsem):
        my = lax.axis_index("i")

        def kv_rdma(src, d, hbm, vm, send_sems, t):
            if src == 0:
                s = hbm.at[:, :, pl.ds(d * HD, HD)]
                dst = vm.at[:, pl.ds(0, SQ), :]
            else:
                s = hbm.at[:, pl.ds(0, WIN), pl.ds(d * HD, HD)]
                dst = vm.at[:, pl.ds(SQ, WIN), :]
            return pltpu.make_async_remote_copy(
                src_ref=s, dst_ref=dst,
                send_sem=send_sems.at[d], recv_sem=kvrecv.at[src, t],
                device_id=(d,), device_id_type=pl.DeviceIdType.MESH,
            )

        def loc_copy(src, hbm, vm, t):
            if src == 0:
                s = hbm.at[:, :, pl.ds(src * HD, HD)]
                dst = vm.at[:, pl.ds(0, SQ), :]
            else:
                s = hbm.at[:, pl.ds(0, WIN), pl.ds(src * HD, HD)]
                dst = vm.at[:, pl.ds(SQ, WIN), :]
            return pltpu.make_async_copy(s, dst, locsem.at[t])

        for src in (0, 1):
            @pl.when(my == src)
            def _(src=src):
                for d in range(N_DEV):
                    if d == src:
                        continue
                    kv_rdma(src, d, k_ref, kg, ksend, 0).start()
                    kv_rdma(src, d, v_ref, vg, vsend, 1).start()
                loc_copy(src, k_ref, kg, 0).start()
                loc_copy(src, v_ref, vg, 1).start()

        q_all = [
            jnp.dot(x_ref[b], wq_ref[...], preferred_element_type=jnp.float32)
            for b in range(B)
        ]

        for src in (0, 1):
            @pl.when(my == src)
            def _(src=src):
                loc_copy(src, k_ref, kg, 0).wait()
                loc_copy(src, v_ref, vg, 1).wait()

            @pl.when(my != src)
            def _(src=src):
                kv_rdma(src, 0, k_ref, kg, ksend, 0).wait_recv()
                kv_rdma(src, 0, v_ref, vg, vsend, 1).wait_recv()

        ii = lax.broadcasted_iota(jnp.int32, (SQ, KVU), 0)
        jj = lax.broadcasted_iota(jnp.int32, (SQ, KVU), 1)
        mask = jnp.abs(ii - jj) <= WIN
        parts = []
        for b in range(B):
            ctxs = []
            for h in range(NH):
                q = q_all[b][:, h * DH:(h + 1) * DH]
                k = kg[b, :, h * DH:(h + 1) * DH]
                s = lax.dot_general(
                    q, k, (((1,), (1,)), ((), ())),
                    preferred_element_type=jnp.float32,
                ) * 0.125
                s = jnp.where(mask, s, -1e9)
                m = jnp.max(s, axis=-1, keepdims=True)
                w = jnp.exp(s - m)
                w = w / jnp.sum(w, axis=-1, keepdims=True)
                ctxs.append(jnp.dot(
                    w, vg[b, :, h * DH:(h + 1) * DH],
                    preferred_element_type=jnp.float32,
                ))
            ctxb = jnp.concatenate(ctxs, axis=1)
            parts.append(jnp.dot(
                ctxb, wo_ref[...], preferred_element_type=jnp.float32,
            ))
        acc[my] = jnp.stack(parts, axis=0)

        def acc_rdma(d, s_slot):
            return pltpu.make_async_remote_copy(
                src_ref=acc.at[s_slot], dst_ref=acc.at[s_slot],
                send_sem=csend.at[d], recv_sem=accrecv.at[s_slot],
                device_id=(d,), device_id_type=pl.DeviceIdType.MESH,
            )

        for d in range(N_DEV):
            @pl.when(my != d)
            def _(d=d):
                acc_rdma(d, my).start()

        for s_ in range(N_DEV):
            @pl.when(my != s_)
            def _(s_=s_):
                acc_rdma(0, s_).wait_recv()

        out_ref[...] = acc[0] + acc[1] + acc[2] + acc[3]

        for src in (0, 1):
            @pl.when(my == src)
            def _(src=src):
                for d in range(N_DEV):
                    if d == src:
                        continue
                    kv_rdma(src, d, k_ref, kg, ksend, 0).wait_send()
                    kv_rdma(src, d, v_ref, vg, vsend, 1).wait_send()
        for d in range(N_DEV):
            @pl.when(my != d)
            def _(d=d):
                acc_rdma(d, my).wait_send()

    return pl.pallas_call(
        body,
        out_shape=jax.ShapeDtypeStruct((B, SQ, DM), jnp.float32),
        in_specs=[
            pl.BlockSpec(memory_space=pltpu.VMEM),
            pl.BlockSpec(memory_space=pltpu.VMEM),
            pl.BlockSpec(memory_space=pltpu.ANY),
            pl.BlockSpec(memory_space=pltpu.ANY),
            pl.BlockSpec(memory_space=pltpu.VMEM),
        ],
        out_specs=pl.BlockSpec(memory_space=pltpu.VMEM),
        scratch_shapes=[
            pltpu.VMEM((B, KVU, HD), jnp.float32),
            pltpu.VMEM((B, KVU, HD), jnp.float32),
            pltpu.VMEM((N_DEV, B, SQ, DM), jnp.float32),
            pltpu.SemaphoreType.DMA((N_DEV,)),
            pltpu.SemaphoreType.DMA((N_DEV,)),
            pltpu.SemaphoreType.DMA((N_DEV,)),
            pltpu.SemaphoreType.DMA((2, 2)),
            pltpu.SemaphoreType.DMA((N_DEV,)),
            pltpu.SemaphoreType.DMA((2,)),
        ],
    )(x, Wq, k2, v2, Wo)


# baseline (device time: 192388 ns/iter reference)
import jax
import jax.numpy as jnp
from jax import lax
from jax.experimental import pallas as pl
from jax.experimental.pallas import tpu as pltpu

N_DEV = 4
B = 2
SQ = 512
DM = 768
NH = 8
DH = 64
HD = NH * DH
WIN = 128
KVU = SQ + WIN


def kernel(x, Wq, K_ext, V_ext, Wo):
    k2 = K_ext.reshape(B, SQ, N_DEV * HD)
    v2 = V_ext.reshape(B, SQ, N_DEV * HD)

    def body(x_ref, wq_ref, k_ref, v_ref, wo_ref, out_ref,
             kg, vg, acc, ksend, vsend, csend, kvrecv, accrecv, locsem):
        my = lax.axis_index("i")

        def kv_rdma(src, d, hbm, vm, send_sems, t):
            if src == 0:
                s = hbm.at[:, :, pl.ds(d * HD, HD)]
                dst = vm.at[:, pl.ds(0, SQ), :]
            else:
                s = hbm.at[:, pl.ds(0, WIN), pl.ds(d * HD, HD)]
                dst = vm.at[:, pl.ds(SQ, WIN), :]
            return pltpu.make_async_remote_copy(
                src_ref=s, dst_ref=dst,
                send_sem=send_sems.at[d], recv_sem=kvrecv.at[src, t],
                device_id=(d,), device_id_type=pl.DeviceIdType.MESH,
            )

        def loc_copy(src, hbm, vm, t):
            if src == 0:
                s = hbm.at[:, :, pl.ds(src * HD, HD)]
                dst = vm.at[:, pl.ds(0, SQ), :]
            else:
                s = hbm.at[:, pl.ds(0, WIN), pl.ds(src * HD, HD)]
                dst = vm.at[:, pl.ds(SQ, WIN), :]
            return pltpu.make_async_copy(s, dst, locsem.at[t])

        for src in (0, 1):
            @pl.when(my == src)
            def _(src=src):
                for d in range(N_DEV):
                    if d == src:
                        continue
                    kv_rdma(src, d, k_ref, kg, ksend, 0).start()
                    kv_rdma(src, d, v_ref, vg, vsend, 1).start()
                loc_copy(src, k_ref, kg, 0).start()
                loc_copy(src, v_ref, vg, 1).start()

        q_all = [
            jnp.dot(x_ref[b], wq_ref[...], preferred_element_type=jnp.float32)
            for b in range(B)
        ]

        for src in (0, 1):
            @pl.when(my == src)
            def _(src=src):
                loc_copy(src, k_ref, kg, 0).wait()
                loc_copy(src, v_ref, vg, 1).wait()

            @pl.when(my != src)
            def _(src=src):
                kv_rdma(src, 0, k_ref, kg, ksend, 0).wait_recv()
                kv_rdma(src, 0, v_ref, vg, vsend, 1).wait_recv()

        ii = lax.broadcasted_iota(jnp.int32, (SQ, KVU), 0)
        jj = lax.broadcasted_iota(jnp.int32, (SQ, KVU), 1)
        mask = jnp.abs(ii - jj) <= WIN
        parts = []
        for b in range(B):
            ctxs = []
            for h in range(NH):
                q = q_all[b][:, h * DH:(h + 1) * DH]
                k = kg[b, :, h * DH:(h + 1) * DH]
                s = lax.dot_general(
                    q, k, (((1,), (1,)), ((), ())),
                    preferred_element_type=jnp.float32,
                ) * 0.125
                s = jnp.where(mask, s, -1e9)
                m = jnp.max(s, axis=-1, keepdims=True)
                w = jnp.exp(s - m)
                w = w / jnp.sum(w, axis=-1, keepdims=True)
                ctxs.append(jnp.dot(
                    w, vg[b, :, h * DH:(h + 1) * DH],
                    preferred_element_type=jnp.float32,
                ))
            ctxb = jnp.concatenate(ctxs, axis=1)
            parts.append(jnp.dot(
                ctxb, wo_ref[...], preferred_element_type=jnp.float32,
            ))
        acc[my] = jnp.stack(parts, axis=0)

        def acc_rdma(d, s_slot):
            return pltpu.make_async_remote_copy(
                src_ref=acc.at[s_slot], dst_ref=acc.at[s_slot],
                send_sem=csend.at[d], recv_sem=accrecv.at[s_slot],
                device_id=(d,), device_id_type=pl.DeviceIdType.MESH,
            )

        for d in range(N_DEV):
            @pl.when(my != d)
            def _(d=d):
                acc_rdma(d, my).start()

        for s_ in range(N_DEV):
            @pl.when(my != s_)
            def _(s_=s_):
                acc_rdma(0, s_).wait_recv()

        out_ref[...] = acc[0] + acc[1] + acc[2] + acc[3]

        for src in (0, 1):
            @pl.when(my == src)
            def _(src=src):
                for d in range(N_DEV):
                    if d == src:
                        continue
                    kv_rdma(src, d, k_ref, kg, ksend, 0).wait_send()
                    kv_rdma(src, d, v_ref, vg, vsend, 1).wait_send()
        for d in range(N_DEV):
            @pl.when(my != d)
            def _(d=d):
                acc_rdma(d, my).wait_send()

    return pl.pallas_call(
        body,
        out_shape=jax.ShapeDtypeStruct((B, SQ, DM), jnp.float32),
        in_specs=[
            pl.BlockSpec(memory_space=pltpu.VMEM),
            pl.BlockSpec(memory_space=pltpu.VMEM),
            pl.BlockSpec(memory_space=pl.ANY),
            pl.BlockSpec(memory_space=pl.ANY),
            pl.BlockSpec(memory_space=pltpu.VMEM),
        ],
        out_specs=pl.BlockSpec(memory_space=pltpu.VMEM),
        scratch_shapes=[
            pltpu.VMEM((B, KVU, HD), jnp.float32),
            pltpu.VMEM((B, KVU, HD), jnp.float32),
            pltpu.VMEM((N_DEV, B, SQ, DM), jnp.float32),
            pltpu.SemaphoreType.DMA((N_DEV,)),
            pltpu.SemaphoreType.DMA((N_DEV,)),
            pltpu.SemaphoreType.DMA((N_DEV,)),
            pltpu.SemaphoreType.DMA((2, 2)),
            pltpu.SemaphoreType.DMA((N_DEV,)),
            pltpu.SemaphoreType.DMA((2,)),
        ],
    )(x, Wq, k2, v2, Wo)


# device time: 106698 ns/iter; 1.8031x vs baseline; 1.8031x over previous
import jax
import jax.numpy as jnp
from jax import lax
from jax.experimental import pallas as pl
from jax.experimental.pallas import tpu as pltpu

N_DEV = 4
B = 2
SQ = 512
DM = 768
NH = 8
DH = 64
HD = NH * DH
WIN = 128
KVU = SQ + WIN
HALF = SQ // 2
QTR = SQ // 4

BF = jnp.bfloat16
F32 = jnp.float32


def kernel(x, Wq, K_ext, V_ext, Wo):
    k2 = K_ext.reshape(B, SQ, N_DEV * HD).astype(BF)
    v2 = V_ext.reshape(B, SQ, N_DEV * HD).astype(BF)

    def body(x_ref, wq_ref, k_ref, v_ref, wo_ref, out_ref,
             kg, vg,
             whsend, hrecv, wqsend, qrecv, ssend, agr1, h2send, agr2,
             ksend, vsend, kvrecv, locsem, csems, crecv):
        my = lax.axis_index("i")

        def kv_rdma(src, d, hbm, vm, send_sems, t):
            if src == 0:
                s = hbm.at[:, :, pl.ds(d * HD, HD)]
                dst = vm.at[:, pl.ds(0, SQ), :]
            else:
                s = hbm.at[:, pl.ds(0, WIN), pl.ds(d * HD, HD)]
                dst = vm.at[:, pl.ds(SQ, WIN), :]
            return pltpu.make_async_remote_copy(
                src_ref=s, dst_ref=dst,
                send_sem=send_sems.at[d], recv_sem=kvrecv.at[src, t],
                device_id=(d,), device_id_type=pl.DeviceIdType.MESH,
            )

        def loc_copy(src, hbm, vm, t):
            if src == 0:
                s = hbm.at[:, :, pl.ds(src * HD, HD)]
                dst = vm.at[:, pl.ds(0, SQ), :]
            else:
                s = hbm.at[:, pl.ds(0, WIN), pl.ds(src * HD, HD)]
                dst = vm.at[:, pl.ds(SQ, WIN), :]
            return pltpu.make_async_copy(s, dst, locsem.at[t])

        for src in (0, 1):
            @pl.when(my == src)
            def _(src=src):
                for d in range(N_DEV):
                    if d == src:
                        continue
                    kv_rdma(src, d, k_ref, kg, ksend, 0).start()
                    kv_rdma(src, d, v_ref, vg, vsend, 1).start()
                loc_copy(src, k_ref, kg, 0).start()
                loc_copy(src, v_ref, vg, 1).start()

        q_all = [
            jnp.dot(x_ref[b], wq_ref[...], preferred_element_type=F32)
            for b in range(B)
        ]

        for src in (0, 1):
            @pl.when(my == src)
            def _(src=src):
                loc_copy(src, k_ref, kg, 0).wait()
                loc_copy(src, v_ref, vg, 1).wait()

            @pl.when(my != src)
            def _(src=src):
                kv_rdma(src, 0, k_ref, kg, ksend, 0).wait_recv()
                kv_rdma(src, 0, v_ref, vg, vsend, 1).wait_recv()

        ii = lax.broadcasted_iota(jnp.int32, (SQ, KVU), 0)
        jj = lax.broadcasted_iota(jnp.int32, (SQ, KVU), 1)
        mask = jnp.abs(ii - jj) <= WIN
        parts = []
        for b in range(B):
            ctxs = []
            for h in range(NH):
                q = q_all[b][:, h * DH:(h + 1) * DH]
                k = kg[b, :, h * DH:(h + 1) * DH].astype(F32)
                s = lax.dot_general(
                    q, k, (((1,), (1,)), ((), ())),
                    preferred_element_type=F32,
                ) * 0.125
                s = jnp.where(mask, s, -1e9)
                m = jnp.max(s, axis=-1, keepdims=True)
                w = jnp.exp(s - m)
                w = w / jnp.sum(w, axis=-1, keepdims=True)
                v = vg[b, :, h * DH:(h + 1) * DH].astype(F32)
                ctxs.append(jnp.dot(w, v, preferred_element_type=F32))
            ctxb = jnp.concatenate(ctxs, axis=1)
            parts.append(jnp.dot(ctxb, wo_ref[...], preferred_element_type=F32))
        partial = jnp.stack(parts, axis=0)

        for src in (0, 1):
            @pl.when(my == src)
            def _(src=src):
                for d in range(N_DEV):
                    if d == src:
                        continue
                    kv_rdma(src, d, k_ref, kg, ksend, 0).wait_send()
                    kv_rdma(src, d, v_ref, vg, vsend, 1).wait_send()

        def xchg(src_ref_, dst_ref_, partner, idx):
            return pltpu.make_async_remote_copy(
                src_ref=src_ref_, dst_ref=dst_ref_,
                send_sem=csems.at[idx], recv_sem=crecv.at[idx],
                device_id=(partner,), device_id_type=pl.DeviceIdType.MESH,
            )

        for pos in range(N_DEV):
            p1 = 3 - pos
            p2 = pos ^ 1
            a = pos >> 1
            bq = pos & 1
            mh = HALF * a
            oh = HALF * (1 - a)
            mq = mh + QTR * bq
            oq = mh + QTR * (1 - bq)

            @pl.when(my == pos)
            def _(p1=p1, p2=p2, mh=mh, oh=oh, mq=mq, oq=oq, bq=bq):
                whsend[...] = partial[:, oh:oh + HALF, :].astype(BF)
                r1 = xchg(whsend, hrecv, p1, 0)
                r1.start()
                r1.wait_recv()
                half = partial[:, mh:mh + HALF, :] + hrecv[...].astype(F32)
                wqsend[...] = half[:, QTR * (1 - bq):QTR * (1 - bq) + QTR, :].astype(BF)
                r2 = xchg(wqsend, qrecv, p2, 1)
                r2.start()
                r2.wait_recv()
                seg = half[:, QTR * bq:QTR * bq + QTR, :] + qrecv[...].astype(F32)
                ssend[...] = seg.astype(BF)
                r3 = xchg(ssend, agr1, p2, 2)
                r3.start()
                r3.wait_recv()
                h2send[:, QTR * bq:QTR * bq + QTR, :] = ssend[...]
                h2send[:, QTR * (1 - bq):QTR * (1 - bq) + QTR, :] = agr1[...]
                r4 = xchg(h2send, agr2, p1, 3)
                r4.start()
                r4.wait_recv()
                out_ref[:, mq:mq + QTR, :] = seg
                out_ref[:, oq:oq + QTR, :] = agr1[...].astype(F32)
                out_ref[:, oh:oh + HALF, :] = agr2[...].astype(F32)
                r1.wait_send()
                r2.wait_send()
                r3.wait_send()
                r4.wait_send()

    return pl.pallas_call(
        body,
        out_shape=jax.ShapeDtypeStruct((B, SQ, DM), F32),
        in_specs=[
            pl.BlockSpec(memory_space=pltpu.VMEM),
            pl.BlockSpec(memory_space=pltpu.VMEM),
            pl.BlockSpec(memory_space=pl.ANY),
            pl.BlockSpec(memory_space=pl.ANY),
            pl.BlockSpec(memory_space=pltpu.VMEM),
        ],
        out_specs=pl.BlockSpec(memory_space=pltpu.VMEM),
        scratch_shapes=[
            pltpu.VMEM((B, KVU, HD), BF),
            pltpu.VMEM((B, KVU, HD), BF),
            pltpu.VMEM((B, HALF, DM), BF),
            pltpu.VMEM((B, HALF, DM), BF),
            pltpu.VMEM((B, QTR, DM), BF),
            pltpu.VMEM((B, QTR, DM), BF),
            pltpu.VMEM((B, QTR, DM), BF),
            pltpu.VMEM((B, QTR, DM), BF),
            pltpu.VMEM((B, HALF, DM), BF),
            pltpu.VMEM((B, HALF, DM), BF),
            pltpu.SemaphoreType.DMA((N_DEV,)),
            pltpu.SemaphoreType.DMA((N_DEV,)),
            pltpu.SemaphoreType.DMA((2, 2)),
            pltpu.SemaphoreType.DMA((2,)),
            pltpu.SemaphoreType.DMA((4,)),
            pltpu.SemaphoreType.DMA((4,)),
        ],
    )(x, Wq, k2, v2, Wo)


# device time: 89768 ns/iter; 2.1432x vs baseline; 1.1886x over previous
import jax
import jax.numpy as jnp
from jax import lax
from jax.experimental import pallas as pl
from jax.experimental.pallas import tpu as pltpu

N_DEV = 4
B = 2
SQ = 512
DM = 768
NH = 8
DH = 64
HD = NH * DH
WIN = 128
KVU = SQ + WIN
HALF = SQ // 2
QTR = SQ // 4

BF = jnp.bfloat16
F32 = jnp.float32


def kernel(x, Wq, K_ext, V_ext, Wo):
    k2 = K_ext.reshape(B, SQ, N_DEV * HD).astype(BF)
    v2 = V_ext.reshape(B, SQ, N_DEV * HD).astype(BF)

    def body(x_ref, wq_ref, k_ref, v_ref, wo_ref, out_ref,
             kg, vg,
             whsend, hrecv, wqsend, qrecv, ssend, agr1, h2send, agr2,
             ksend, vsend, kvrecv, locsem, csems, crecv):
        my = lax.axis_index("i")

        def kv_rdma(src, d, bb, hbm, vm, send_sems, t):
            if src == 0:
                s = hbm.at[bb, :, pl.ds(d * HD, HD)]
                dst = vm.at[bb, pl.ds(0, SQ), :]
            else:
                s = hbm.at[bb, pl.ds(0, WIN), pl.ds(d * HD, HD)]
                dst = vm.at[bb, pl.ds(SQ, WIN), :]
            return pltpu.make_async_remote_copy(
                src_ref=s, dst_ref=dst,
                send_sem=send_sems.at[d, bb], recv_sem=kvrecv.at[src, t, bb],
                device_id=(d,), device_id_type=pl.DeviceIdType.MESH,
            )

        def loc_copy(src, bb, hbm, vm, t):
            if src == 0:
                s = hbm.at[bb, :, pl.ds(src * HD, HD)]
                dst = vm.at[bb, pl.ds(0, SQ), :]
            else:
                s = hbm.at[bb, pl.ds(0, WIN), pl.ds(src * HD, HD)]
                dst = vm.at[bb, pl.ds(SQ, WIN), :]
            return pltpu.make_async_copy(s, dst, locsem.at[t, bb])

        for src in (0, 1):
            @pl.when(my == src)
            def _(src=src):
                for bb in range(B):
                    for d in range(N_DEV):
                        if d == src:
                            continue
                        kv_rdma(src, d, bb, k_ref, kg, ksend, 0).start()
                        kv_rdma(src, d, bb, v_ref, vg, vsend, 1).start()
                    loc_copy(src, bb, k_ref, kg, 0).start()
                    loc_copy(src, bb, v_ref, vg, 1).start()

        q_all = [
            jnp.dot(x_ref[b], wq_ref[...], preferred_element_type=F32)
            for b in range(B)
        ]

        i1 = lax.broadcasted_iota(jnp.int32, (HALF, 384), 0)
        j1 = lax.broadcasted_iota(jnp.int32, (HALF, 384), 1)
        mask1 = jnp.abs(i1 - j1) <= WIN
        i2 = lax.broadcasted_iota(jnp.int32, (HALF, SQ), 0)
        j2 = lax.broadcasted_iota(jnp.int32, (HALF, SQ), 1)
        mask2 = jnp.abs(i2 + WIN - j2) <= WIN

        def wait_kv(bb):
            for src in (0, 1):
                @pl.when(my == src)
                def _(src=src):
                    loc_copy(src, bb, k_ref, kg, 0).wait()
                    loc_copy(src, bb, v_ref, vg, 1).wait()

                @pl.when(my != src)
                def _(src=src):
                    kv_rdma(src, 0, bb, k_ref, kg, ksend, 0).wait_recv()
                    kv_rdma(src, 0, bb, v_ref, vg, vsend, 1).wait_recv()

        def sm_block(q, k, v, msk):
            s = lax.dot_general(
                q, k, (((1,), (1,)), ((), ())), preferred_element_type=F32,
            ) * 0.125
            s = jnp.where(msk, s, -1e9)
            m = jnp.max(s, axis=-1, keepdims=True)
            w = jnp.exp(s - m)
            w = w / jnp.sum(w, axis=-1, keepdims=True)
            return jnp.dot(w, v, preferred_element_type=F32)

        def attn_batch(bb):
            c1s, c2s = [], []
            for h in range(NH):
                cols = pl.ds(h * DH, DH)
                k1 = kg[bb, 0:384, cols].astype(F32)
                v1 = vg[bb, 0:384, cols].astype(F32)
                k2b = kg[bb, WIN:KVU, cols].astype(F32)
                v2b = vg[bb, WIN:KVU, cols].astype(F32)
                qh = q_all[bb][:, h * DH:(h + 1) * DH]
                c1s.append(sm_block(qh[0:HALF], k1, v1, mask1))
                c2s.append(sm_block(qh[HALF:SQ], k2b, v2b, mask2))
            c1 = jnp.concatenate(c1s, axis=1)
            c2 = jnp.concatenate(c2s, axis=1)
            return jnp.concatenate([
                jnp.dot(c1, wo_ref[...], preferred_element_type=F32),
                jnp.dot(c2, wo_ref[...], preferred_element_type=F32),
            ], axis=0)

        def xchg(src_ref_, dst_ref_, partner, step, bb):
            return pltpu.make_async_remote_copy(
                src_ref=src_ref_, dst_ref=dst_ref_,
                send_sem=csems.at[step, bb], recv_sem=crecv.at[step, bb],
                device_id=(partner,), device_id_type=pl.DeviceIdType.MESH,
            )

        def pos_params(pos):
            p1 = 3 - pos
            p2 = pos ^ 1
            a = pos >> 1
            bq = pos & 1
            mh = HALF * a
            oh = HALF * (1 - a)
            mq = mh + QTR * bq
            oq = mh + QTR * (1 - bq)
            return p1, p2, bq, mh, oh, mq, oq

        wait_kv(0)
        partial0 = attn_batch(0)
        for pos in range(N_DEV):
            p1, p2, bq, mh, oh, mq, oq = pos_params(pos)

            @pl.when(my == pos)
            def _(p1=p1, oh=oh):
                whsend[0] = partial0[oh:oh + HALF, :].astype(BF)
                xchg(whsend.at[0], hrecv.at[0], p1, 0, 0).start()

        wait_kv(1)
        partial1 = attn_batch(1)
        partials = (partial0, partial1)

        for src in (0, 1):
            @pl.when(my == src)
            def _(src=src):
                for bb in range(B):
                    for d in range(N_DEV):
                        if d == src:
                            continue
                        kv_rdma(src, d, bb, k_ref, kg, ksend, 0).wait_send()
                        kv_rdma(src, d, bb, v_ref, vg, vsend, 1).wait_send()

        for pos in range(N_DEV):
            p1, p2, bq, mh, oh, mq, oq = pos_params(pos)

            @pl.when(my == pos)
            def _(p1=p1, p2=p2, bq=bq, mh=mh, oh=oh, mq=mq, oq=oq):
                whsend[1] = partial1[oh:oh + HALF, :].astype(BF)
                xchg(whsend.at[1], hrecv.at[1], p1, 0, 1).start()

                halves = []
                for bb in range(B):
                    xchg(whsend.at[bb], hrecv.at[bb], p1, 0, bb).wait_recv()
                    half = partials[bb][mh:mh + HALF, :] + hrecv[bb].astype(F32)
                    halves.append(half)
                    wqsend[bb] = half[QTR * (1 - bq):QTR * (1 - bq) + QTR, :].astype(BF)
                    xchg(wqsend.at[bb], qrecv.at[bb], p2, 1, bb).start()

                segs = []
                for bb in range(B):
                    xchg(wqsend.at[bb], qrecv.at[bb], p2, 1, bb).wait_recv()
                    seg = halves[bb][QTR * bq:QTR * bq + QTR, :] + qrecv[bb].astype(F32)
                    segs.append(seg)
                    ssend[bb] = seg.astype(BF)
                    xchg(ssend.at[bb], agr1.at[bb], p2, 2, bb).start()

                for bb in range(B):
                    xchg(ssend.at[bb], agr1.at[bb], p2, 2, bb).wait_recv()
                    h2send[bb, QTR * bq:QTR * bq + QTR, :] = ssend[bb]
                    h2send[bb, QTR * (1 - bq):QTR * (1 - bq) + QTR, :] = agr1[bb]
                    xchg(h2send.at[bb], agr2.at[bb], p1, 3, bb).start()
                    out_ref[bb, mq:mq + QTR, :] = segs[bb]
                    out_ref[bb, oq:oq + QTR, :] = agr1[bb].astype(F32)

                for bb in range(B):
                    xchg(h2send.at[bb], agr2.at[bb], p1, 3, bb).wait_recv()
                    out_ref[bb, oh:oh + HALF, :] = agr2[bb].astype(F32)

                for bb in range(B):
                    for step, (sref, dref, pp) in enumerate((
                        (whsend, hrecv, p1), (wqsend, qrecv, p2),
                        (ssend, agr1, p2), (h2send, agr2, p1),
                    )):
                        xchg(sref.at[bb], dref.at[bb], pp, step, bb).wait_send()

    return pl.pallas_call(
        body,
        out_shape=jax.ShapeDtypeStruct((B, SQ, DM), F32),
        in_specs=[
            pl.BlockSpec(memory_space=pltpu.VMEM),
            pl.BlockSpec(memory_space=pltpu.VMEM),
            pl.BlockSpec(memory_space=pl.ANY),
            pl.BlockSpec(memory_space=pl.ANY),
            pl.BlockSpec(memory_space=pltpu.VMEM),
        ],
        out_specs=pl.BlockSpec(memory_space=pltpu.VMEM),
        scratch_shapes=[
            pltpu.VMEM((B, KVU, HD), BF),
            pltpu.VMEM((B, KVU, HD), BF),
            pltpu.VMEM((B, HALF, DM), BF),
            pltpu.VMEM((B, HALF, DM), BF),
            pltpu.VMEM((B, QTR, DM), BF),
            pltpu.VMEM((B, QTR, DM), BF),
            pltpu.VMEM((B, QTR, DM), BF),
            pltpu.VMEM((B, QTR, DM), BF),
            pltpu.VMEM((B, HALF, DM), BF),
            pltpu.VMEM((B, HALF, DM), BF),
            pltpu.SemaphoreType.DMA((N_DEV, B)),
            pltpu.SemaphoreType.DMA((N_DEV, B)),
            pltpu.SemaphoreType.DMA((2, 2, B)),
            pltpu.SemaphoreType.DMA((2, B)),
            pltpu.SemaphoreType.DMA((4, B)),
            pltpu.SemaphoreType.DMA((4, B)),
        ],
    )(x, Wq, k2, v2, Wo)
